# baseline (speedup 1.0000x reference)
import jax
import jax.numpy as jnp
from jax import lax
from jax.experimental import pallas as pl
from jax.experimental.pallas import tpu as pltpu
from jax.experimental.pallas import tpu_sc as plsc

N_NODES = 10000
DIM = 128
N_EDGES = 320000
NUM_CORES = 2
NUM_SUBCORES = 16
LANES = 16

CHUNK = 128
NCH = 2560
E_PAD = NCH * CHUNK
NPT = NCH // NUM_SUBCORES
ROWS_A = 624
ROWS_LAST = N_NODES - ROWS_A * (NUM_SUBCORES - 1)


def _matmul_body(x_ref, w_ref, o_ref):
    o_ref[...] = lax.dot_general(
        x_ref[...], w_ref[...], (((1,), (1,)), ((), ())),
        preferred_element_type=jnp.float32)


def _h0_matmul(x, W1):
    return pl.pallas_call(
        _matmul_body,
        grid=(10,),
        in_specs=[pl.BlockSpec((1000, DIM), lambda i: (i, 0)),
                  pl.BlockSpec((DIM, DIM), lambda i: (0, 0))],
        out_specs=pl.BlockSpec((1000, DIM), lambda i: (i, 0)),
        out_shape=jax.ShapeDtypeStruct((N_NODES, DIM), jnp.float32),
    )(x, W1)


def _spmm_body(h0_hbm, dst_hbm, src_hbm, val_hbm, zeros_hbm, out_hbm,
               dst_ring, src_ring, val_ring, rows, acc_sh,
               gsem, ssem, isem, isem0):
    c = lax.axis_index("c")
    s = lax.axis_index("s")
    row0 = s * ROWS_A
    last = NUM_SUBCORES - 1
    ebase = c * E_PAD + s * NPT * CHUNK

    def issue_idx(chunk_i, slot, sem):
        off = ebase + chunk_i * CHUNK
        pltpu.async_copy(dst_hbm.at[pl.ds(off, CHUNK)], dst_ring.at[slot], sem)
        pltpu.async_copy(src_hbm.at[pl.ds(off, CHUNK)], src_ring.at[slot], sem)
        pltpu.async_copy(val_hbm.at[pl.ds(off, CHUNK)], val_ring.at[slot], sem)

    def wait_idx(sem):
        pltpu.make_async_copy(dst_hbm.at[pl.ds(0, CHUNK)], dst_ring.at[0], sem).wait()
        pltpu.make_async_copy(src_hbm.at[pl.ds(0, CHUNK)], src_ring.at[0], sem).wait()
        pltpu.make_async_copy(val_hbm.at[pl.ds(0, CHUNK)], val_ring.at[0], sem).wait()

    def scale_rows(p):
        for g in range(CHUNK // LANES):
            v16 = val_ring[p, pl.ds(g * LANES, LANES)]
            for l in range(LANES):
                e = g * LANES + l
                vv = jnp.broadcast_to(v16[l], (LANES,))
                for j in range(DIM // LANES):
                    sl = pl.ds(j * LANES, LANES)
                    rows[e, sl] = rows[e, sl] * vv

    def chunk(i, p):
        wait_idx(isem)
        issue_idx(i + 1, 1 - p, isem)
        pltpu.async_copy(h0_hbm.at[src_ring.at[p]], rows, gsem)
        pltpu.make_async_copy(h0_hbm.at[src_ring.at[0]], rows, gsem).wait()
        scale_rows(p)
        pltpu.async_copy(rows, acc_sh.at[dst_ring.at[p]], ssem, add=True)
        pltpu.make_async_copy(rows, acc_sh.at[dst_ring.at[0]], ssem).wait()

    issue_idx(0, 0, isem)

    @pl.when(s < last)
    def _():
        pltpu.sync_copy(zeros_hbm.at[pl.ds(0, ROWS_A)],
                        acc_sh.at[pl.ds(row0, ROWS_A)])

    @pl.when(s == last)
    def _():
        pltpu.sync_copy(zeros_hbm, acc_sh.at[pl.ds(last * ROWS_A, ROWS_LAST)])

    plsc.subcore_barrier()

    def body(t, carry):
        chunk(2 * t, 0)
        chunk(2 * t + 1, 1)
        return carry

    lax.fori_loop(0, NPT // 2, body, 0)

    wait_idx(isem)
    plsc.subcore_barrier()

    @pl.when(s < last)
    def _():
        pltpu.sync_copy(acc_sh.at[pl.ds(row0, ROWS_A)],
                        out_hbm.at[c, pl.ds(row0, ROWS_A)])

    @pl.when(s == last)
    def _():
        pltpu.sync_copy(acc_sh.at[pl.ds(last * ROWS_A, ROWS_LAST)],
                        out_hbm.at[c, pl.ds(last * ROWS_A, ROWS_LAST)])


def _spmm_both(h0, dst_all, src_all, val_all, zeros):
    mesh = plsc.VectorSubcoreMesh(core_axis_name="c", subcore_axis_name="s")
    return pl.kernel(
        _spmm_body,
        out_type=jax.ShapeDtypeStruct((NUM_CORES, N_NODES, DIM), jnp.float32),
        mesh=mesh,
        scratch_types=[
            pltpu.VMEM((2, CHUNK), jnp.int32),
            pltpu.VMEM((2, CHUNK), jnp.int32),
            pltpu.VMEM((2, CHUNK), jnp.float32),
            pltpu.VMEM((CHUNK, DIM), jnp.float32),
            pltpu.VMEM_SHARED((N_NODES, DIM), jnp.float32),
            pltpu.SemaphoreType.DMA,
            pltpu.SemaphoreType.DMA,
            pltpu.SemaphoreType.DMA,
            pltpu.SemaphoreType.DMA,
        ],
    )(h0, dst_all, src_all, val_all, zeros)


def _pad_edges(a):
    return jnp.concatenate([a, jnp.zeros((E_PAD - N_EDGES,), a.dtype)])


def kernel(x, adj1_indices, adj1_values, adj2_indices, adj2_values, W1):
    h0 = _h0_matmul(x, W1)
    i1 = adj1_indices.astype(jnp.int32)
    i2 = adj2_indices.astype(jnp.int32)
    tail = jnp.zeros((CHUNK,), jnp.int32)
    dst_all = jnp.concatenate([_pad_edges(i1[0]), _pad_edges(i2[0]), tail])
    src_all = jnp.concatenate([_pad_edges(i1[1]), _pad_edges(i2[1]), tail])
    val_all = jnp.concatenate([_pad_edges(adj1_values), _pad_edges(adj2_values),
                               tail.astype(jnp.float32)])
    zeros = jnp.zeros((ROWS_LAST, DIM), jnp.float32)
    hops = _spmm_both(h0, dst_all, src_all, val_all, zeros)
    return jnp.concatenate([h0, hops[0], hops[1]], axis=1)


# speedup vs baseline: 1.0955x; 1.0955x over previous
import jax
import jax.numpy as jnp
from jax import lax
from jax.experimental import pallas as pl
from jax.experimental.pallas import tpu as pltpu
from jax.experimental.pallas import tpu_sc as plsc

N_NODES = 10000
DIM = 128
N_EDGES = 320000
NUM_CORES = 2
NUM_SUBCORES = 16
LANES = 16

CHUNK = 128
NCH = 2560
E_PAD = NCH * CHUNK
NPT = NCH // NUM_SUBCORES
ROWS_A = 624
ROWS_LAST = N_NODES - ROWS_A * (NUM_SUBCORES - 1)


def _matmul_body(x_ref, w_ref, o_ref):
    o_ref[...] = lax.dot_general(
        x_ref[...], w_ref[...], (((1,), (1,)), ((), ())),
        preferred_element_type=jnp.float32)


def _h0_matmul(x, W1):
    return pl.pallas_call(
        _matmul_body,
        grid=(10,),
        in_specs=[pl.BlockSpec((1000, DIM), lambda i: (i, 0)),
                  pl.BlockSpec((DIM, DIM), lambda i: (0, 0))],
        out_specs=pl.BlockSpec((1000, DIM), lambda i: (i, 0)),
        out_shape=jax.ShapeDtypeStruct((N_NODES, DIM), jnp.float32),
    )(x, W1)


def _spmm_body(h0_hbm, dst_hbm, src_hbm, val_hbm, zeros_hbm, out_hbm,
               dst_ring, src_ring, val_ring, rows, acc_sh,
               gsem, ssem, isem, isem0):
    c = lax.axis_index("c")
    s = lax.axis_index("s")
    row0 = s * ROWS_A
    last = NUM_SUBCORES - 1
    ebase = c * E_PAD + s * NPT * CHUNK

    def issue_idx(chunk_i, slot, vslot, sem):
        off = ebase + chunk_i * CHUNK
        sl = pl.ds(slot * CHUNK, CHUNK)
        pltpu.async_copy(dst_hbm.at[pl.ds(off, CHUNK)], dst_ring.at[sl], sem)
        pltpu.async_copy(src_hbm.at[pl.ds(off, CHUNK)], src_ring.at[sl], sem)
        pltpu.async_copy(val_hbm.at[pl.ds(off, CHUNK)], val_ring.at[vslot], sem)

    def wait_idx(sem):
        sl = pl.ds(0, CHUNK)
        pltpu.make_async_copy(dst_hbm.at[pl.ds(0, CHUNK)], dst_ring.at[sl], sem).wait()
        pltpu.make_async_copy(src_hbm.at[pl.ds(0, CHUNK)], src_ring.at[sl], sem).wait()
        pltpu.make_async_copy(val_hbm.at[pl.ds(0, CHUNK)], val_ring.at[0], sem).wait()

    def issue_gather(slot, buf):
        pltpu.async_copy(h0_hbm.at[src_ring.at[pl.ds(slot * CHUNK, CHUNK)]],
                         rows.at[buf], gsem)

    def wait_gather():
        pltpu.make_async_copy(h0_hbm.at[src_ring.at[pl.ds(0, CHUNK)]],
                              rows.at[0], gsem).wait()

    def issue_scatter(slot, buf):
        pltpu.async_copy(rows.at[buf],
                         acc_sh.at[dst_ring.at[pl.ds(slot * CHUNK, CHUNK)]],
                         ssem, add=True)

    def wait_scatter():
        pltpu.make_async_copy(rows.at[0],
                              acc_sh.at[dst_ring.at[pl.ds(0, CHUNK)]],
                              ssem).wait()

    def scale_rows(p):
        for g in range(CHUNK // LANES):
            v16 = val_ring[p, pl.ds(g * LANES, LANES)]
            for l in range(LANES):
                e = g * LANES + l
                vv = jnp.broadcast_to(v16[l], (LANES,))
                for j in range(DIM // LANES):
                    sl = pl.ds(j * LANES, LANES)
                    rows[p, e, sl] = rows[p, e, sl] * vv

    zoff = NUM_CORES * E_PAD
    pltpu.sync_copy(dst_hbm.at[pl.ds(zoff, CHUNK)],
                    dst_ring.at[pl.ds(3 * CHUNK, CHUNK)])
    pltpu.sync_copy(zeros_hbm.at[pl.ds(0, CHUNK)], rows.at[1])

    issue_idx(0, 0, 0, isem0)
    issue_idx(1, 1, 1, isem)

    @pl.when(s < last)
    def _():
        pltpu.sync_copy(zeros_hbm.at[pl.ds(0, ROWS_A)],
                        acc_sh.at[pl.ds(row0, ROWS_A)])

    @pl.when(s == last)
    def _():
        pltpu.sync_copy(zeros_hbm, acc_sh.at[pl.ds(last * ROWS_A, ROWS_LAST)])

    wait_idx(isem0)
    issue_gather(0, 0)

    plsc.subcore_barrier()

    issue_scatter(3, 1)

    def chunk(i, p):
        wait_gather()

        wait_scatter()

        wait_idx(isem)
        issue_gather(lax.rem(i + 1, 4), 1 - p)

        scale_rows(p)

        issue_idx(i + 2, lax.rem(i + 2, 4), p, isem)

        issue_scatter(lax.rem(i, 4), p)

    def body(t, carry):
        chunk(2 * t, 0)
        chunk(2 * t + 1, 1)
        return carry

    lax.fori_loop(0, NPT // 2, body, 0)

    wait_scatter()
    wait_gather()
    wait_idx(isem)
    plsc.subcore_barrier()

    @pl.when(s < last)
    def _():
        pltpu.sync_copy(acc_sh.at[pl.ds(row0, ROWS_A)],
                        out_hbm.at[c, pl.ds(row0, ROWS_A)])

    @pl.when(s == last)
    def _():
        pltpu.sync_copy(acc_sh.at[pl.ds(last * ROWS_A, ROWS_LAST)],
                        out_hbm.at[c, pl.ds(last * ROWS_A, ROWS_LAST)])


def _spmm_both(h0, dst_all, src_all, val_all, zeros):
    mesh = plsc.VectorSubcoreMesh(core_axis_name="c", subcore_axis_name="s")
    return pl.kernel(
        _spmm_body,
        out_type=jax.ShapeDtypeStruct((NUM_CORES, N_NODES, DIM), jnp.float32),
        mesh=mesh,
        scratch_types=[
            pltpu.VMEM((4 * CHUNK,), jnp.int32),
            pltpu.VMEM((4 * CHUNK,), jnp.int32),
            pltpu.VMEM((2, CHUNK), jnp.float32),
            pltpu.VMEM((2, CHUNK, DIM), jnp.float32),
            pltpu.VMEM_SHARED((N_NODES, DIM), jnp.float32),
            pltpu.SemaphoreType.DMA,
            pltpu.SemaphoreType.DMA,
            pltpu.SemaphoreType.DMA,
            pltpu.SemaphoreType.DMA,
        ],
    )(h0, dst_all, src_all, val_all, zeros)


def _pad_edges(a):
    return jnp.concatenate([a, jnp.zeros((E_PAD - N_EDGES,), a.dtype)])


def kernel(x, adj1_indices, adj1_values, adj2_indices, adj2_values, W1):
    h0 = _h0_matmul(x, W1)
    i1 = adj1_indices.astype(jnp.int32)
    i2 = adj2_indices.astype(jnp.int32)
    tail = jnp.zeros((2 * CHUNK,), jnp.int32)
    dst_all = jnp.concatenate([_pad_edges(i1[0]), _pad_edges(i2[0]), tail])
    src_all = jnp.concatenate([_pad_edges(i1[1]), _pad_edges(i2[1]), tail])
    val_all = jnp.concatenate([_pad_edges(adj1_values), _pad_edges(adj2_values),
                               tail.astype(jnp.float32)])
    zeros = jnp.zeros((ROWS_LAST, DIM), jnp.float32)
    hops = _spmm_both(h0, dst_all, src_all, val_all, zeros)
    return jnp.concatenate([h0, hops[0], hops[1]], axis=1)


# speedup vs baseline: 1.0980x; 1.0023x over previous
import jax
import jax.numpy as jnp
from jax import lax
from jax.experimental import pallas as pl
from jax.experimental.pallas import tpu as pltpu
from jax.experimental.pallas import tpu_sc as plsc

N_NODES = 10000
DIM = 128
N_EDGES = 320000
NUM_CORES = 2
NUM_SUBCORES = 16
LANES = 16

CHUNK = 64
SLOT = 128
NCH = 5120
E_PAD = NCH * CHUNK
NPT = NCH // NUM_SUBCORES
ROWS_A = 624
ROWS_LAST = N_NODES - ROWS_A * (NUM_SUBCORES - 1)


def _matmul_body(x_ref, w_ref, o_ref):
    o_ref[...] = lax.dot_general(
        x_ref[...], w_ref[...], (((1,), (1,)), ((), ())),
        preferred_element_type=jnp.float32)


def _h0_matmul(x, W1):
    return pl.pallas_call(
        _matmul_body,
        grid=(10,),
        in_specs=[pl.BlockSpec((1000, DIM), lambda i: (i, 0)),
                  pl.BlockSpec((DIM, DIM), lambda i: (0, 0))],
        out_specs=pl.BlockSpec((1000, DIM), lambda i: (i, 0)),
        out_shape=jax.ShapeDtypeStruct((N_NODES, DIM), jnp.float32),
    )(x, W1)


def _spmm_body(h0_hbm, dst_hbm, src_hbm, val_hbm, zeros_hbm, out_hbm,
               dst_ring, src_ring, val_ring, rows, acc_sh,
               gsem, ssem, isem, isem0):
    c = lax.axis_index("c")
    s = lax.axis_index("s")
    row0 = s * ROWS_A
    last = NUM_SUBCORES - 1
    ebase = c * E_PAD + s * NPT * CHUNK

    def issue_idx(chunk_i, slot, vslot, sem):
        off = ebase + chunk_i * CHUNK
        sl = pl.ds(slot * SLOT, CHUNK)
        pltpu.async_copy(dst_hbm.at[pl.ds(off, CHUNK)], dst_ring.at[sl], sem)
        pltpu.async_copy(src_hbm.at[pl.ds(off, CHUNK)], src_ring.at[sl], sem)
        pltpu.async_copy(val_hbm.at[pl.ds(off, CHUNK)],
                         val_ring.at[vslot, pl.ds(0, CHUNK)], sem)

    def wait_idx(sem):
        sl = pl.ds(0, CHUNK)
        pltpu.make_async_copy(dst_hbm.at[pl.ds(0, CHUNK)], dst_ring.at[sl], sem).wait()
        pltpu.make_async_copy(src_hbm.at[pl.ds(0, CHUNK)], src_ring.at[sl], sem).wait()
        pltpu.make_async_copy(val_hbm.at[pl.ds(0, CHUNK)],
                              val_ring.at[0, pl.ds(0, CHUNK)], sem).wait()

    def issue_gather(slot, buf):
        pltpu.async_copy(h0_hbm.at[src_ring.at[pl.ds(slot * SLOT, CHUNK)]],
                         rows.at[buf], gsem)

    def wait_gather():
        pltpu.make_async_copy(h0_hbm.at[src_ring.at[pl.ds(0, CHUNK)]],
                              rows.at[0], gsem).wait()

    def issue_scatter(slot, buf):
        pltpu.async_copy(rows.at[buf],
                         acc_sh.at[dst_ring.at[pl.ds(slot * SLOT, CHUNK)]],
                         ssem, add=True)

    def wait_scatter():
        pltpu.make_async_copy(rows.at[0],
                              acc_sh.at[dst_ring.at[pl.ds(0, CHUNK)]],
                              ssem).wait()

    def scale_rows(p):
        for g in range(CHUNK // LANES):
            v16 = val_ring[p, pl.ds(g * LANES, LANES)]
            for l in range(LANES):
                e = g * LANES + l
                vv = jnp.broadcast_to(v16[l], (LANES,))
                for j in range(DIM // LANES):
                    sl = pl.ds(j * LANES, LANES)
                    rows[p, e, sl] = rows[p, e, sl] * vv

    zoff = NUM_CORES * E_PAD
    pltpu.sync_copy(dst_hbm.at[pl.ds(zoff, CHUNK)],
                    dst_ring.at[pl.ds(3 * SLOT, CHUNK)])
    pltpu.sync_copy(zeros_hbm.at[pl.ds(0, CHUNK)], rows.at[1])

    issue_idx(0, 0, 0, isem0)
    issue_idx(1, 1, 1, isem)

    @pl.when(s < last)
    def _():
        pltpu.sync_copy(zeros_hbm.at[pl.ds(0, ROWS_A)],
                        acc_sh.at[pl.ds(row0, ROWS_A)])

    @pl.when(s == last)
    def _():
        pltpu.sync_copy(zeros_hbm, acc_sh.at[pl.ds(last * ROWS_A, ROWS_LAST)])

    wait_idx(isem0)
    issue_gather(0, 0)

    plsc.subcore_barrier()

    issue_scatter(3, 1)

    def chunk(i, p):
        wait_gather()

        wait_scatter()

        wait_idx(isem)
        issue_gather(lax.rem(i + 1, 4), 1 - p)

        scale_rows(p)

        issue_scatter(lax.rem(i, 4), p)

        issue_idx(i + 2, lax.rem(i + 2, 4), p, isem)

    def body(t, carry):
        chunk(2 * t, 0)
        chunk(2 * t + 1, 1)
        return carry

    lax.fori_loop(0, NPT // 2, body, 0)

    wait_scatter()
    wait_gather()
    wait_idx(isem)
    plsc.subcore_barrier()

    @pl.when(s < last)
    def _():
        pltpu.sync_copy(acc_sh.at[pl.ds(row0, ROWS_A)],
                        out_hbm.at[c, pl.ds(row0, ROWS_A)])

    @pl.when(s == last)
    def _():
        pltpu.sync_copy(acc_sh.at[pl.ds(last * ROWS_A, ROWS_LAST)],
                        out_hbm.at[c, pl.ds(last * ROWS_A, ROWS_LAST)])


def _spmm_both(h0, dst_all, src_all, val_all, zeros):
    mesh = plsc.VectorSubcoreMesh(core_axis_name="c", subcore_axis_name="s")
    return pl.kernel(
        _spmm_body,
        out_type=jax.ShapeDtypeStruct((NUM_CORES, N_NODES, DIM), jnp.float32),
        mesh=mesh,
        scratch_types=[
            pltpu.VMEM((4 * SLOT,), jnp.int32),
            pltpu.VMEM((4 * SLOT,), jnp.int32),
            pltpu.VMEM((2, SLOT), jnp.float32),
            pltpu.VMEM((2, CHUNK, DIM), jnp.float32),
            pltpu.VMEM_SHARED((N_NODES, DIM), jnp.float32),
            pltpu.SemaphoreType.DMA,
            pltpu.SemaphoreType.DMA,
            pltpu.SemaphoreType.DMA,
            pltpu.SemaphoreType.DMA,
        ],
    )(h0, dst_all, src_all, val_all, zeros)


def _pad_edges(a):
    return jnp.concatenate([a, jnp.zeros((E_PAD - N_EDGES,), a.dtype)])


def kernel(x, adj1_indices, adj1_values, adj2_indices, adj2_values, W1):
    h0 = _h0_matmul(x, W1)
    i1 = adj1_indices.astype(jnp.int32)
    i2 = adj2_indices.astype(jnp.int32)
    tail = jnp.zeros((2 * CHUNK,), jnp.int32)
    dst_all = jnp.concatenate([_pad_edges(i1[0]), _pad_edges(i2[0]), tail])
    src_all = jnp.concatenate([_pad_edges(i1[1]), _pad_edges(i2[1]), tail])
    val_all = jnp.concatenate([_pad_edges(adj1_values), _pad_edges(adj2_values),
                               tail.astype(jnp.float32)])
    zeros = jnp.zeros((ROWS_LAST, DIM), jnp.float32)
    hops = _spmm_both(h0, dst_all, src_all, val_all, zeros)
    return jnp.concatenate([h0, hops[0], hops[1]], axis=1)


# speedup vs baseline: 1.2912x; 1.1759x over previous
import jax
import jax.numpy as jnp
from jax import lax
from jax.experimental import pallas as pl
from jax.experimental.pallas import tpu as pltpu
from jax.experimental.pallas import tpu_sc as plsc

N_NODES = 10000
DIM = 128
N_EDGES = 320000
NUM_CORES = 2
NUM_SUBCORES = 16
LANES = 16

CHUNK = 128
NCH = 2560
E_PAD = NCH * CHUNK
NPT = NCH // NUM_SUBCORES
ROWS_A = 624
ROWS_LAST = N_NODES - ROWS_A * (NUM_SUBCORES - 1)


def _matmul_body(x_ref, w_ref, o_ref):
    o_ref[...] = lax.dot_general(
        x_ref[...], w_ref[...], (((1,), (1,)), ((), ())),
        preferred_element_type=jnp.float32)


def _h0_matmul(x, W1):
    return pl.pallas_call(
        _matmul_body,
        grid=(10,),
        in_specs=[pl.BlockSpec((1000, DIM), lambda i: (i, 0)),
                  pl.BlockSpec((DIM, DIM), lambda i: (0, 0))],
        out_specs=pl.BlockSpec((1000, DIM), lambda i: (i, 0)),
        out_shape=jax.ShapeDtypeStruct((N_NODES, DIM), jnp.float32),
    )(x, W1)


def _spmm_body(h0_hbm, dst_hbm, src_hbm, val_hbm, zeros_hbm, out_hbm,
               dst_ring, src_ring, val_ring, rows, acc_sh,
               gsem0, gsem1, ssem0, ssem1, isem0, isem1, isem2, isem3):
    c = lax.axis_index("c")
    s = lax.axis_index("s")
    row0 = s * ROWS_A
    last = NUM_SUBCORES - 1
    isems = [isem0, isem1, isem2, isem3]
    gsems = [gsem0, gsem1]
    ssems = [ssem0, ssem1]
    ebase = c * E_PAD + s * NPT * CHUNK

    def issue_idx(chunk_i, slot, sem):
        off = ebase + chunk_i * CHUNK
        sl = pl.ds(slot * CHUNK, CHUNK)
        pltpu.async_copy(dst_hbm.at[pl.ds(off, CHUNK)], dst_ring.at[sl], sem)
        pltpu.async_copy(src_hbm.at[pl.ds(off, CHUNK)], src_ring.at[sl], sem)
        pltpu.async_copy(val_hbm.at[pl.ds(off, CHUNK)], val_ring.at[sl], sem)

    def wait_idx(sem):
        sl = pl.ds(0, CHUNK)
        pltpu.make_async_copy(dst_hbm.at[pl.ds(0, CHUNK)], dst_ring.at[sl], sem).wait()
        pltpu.make_async_copy(src_hbm.at[pl.ds(0, CHUNK)], src_ring.at[sl], sem).wait()
        pltpu.make_async_copy(val_hbm.at[pl.ds(0, CHUNK)], val_ring.at[sl], sem).wait()

    def wait_gather(sem):
        pltpu.make_async_copy(h0_hbm.at[src_ring.at[pl.ds(0, CHUNK)]], rows.at[0], sem).wait()

    def wait_scatter(sem):
        pltpu.make_async_copy(rows.at[0], acc_sh.at[dst_ring.at[pl.ds(0, CHUNK)]], sem).wait()

    def scale_rows(p, k):
        def body(g, carry):
            e0 = g * LANES
            v16 = val_ring[pl.ds(k * CHUNK + e0, LANES)]
            for l in range(LANES):
                vv = jnp.broadcast_to(v16[l], (LANES,))
                for j in range(DIM // LANES):
                    sl = pl.ds(j * LANES, LANES)
                    rows[p, e0 + l, sl] = rows[p, e0 + l, sl] * vv
            return carry
        lax.fori_loop(0, CHUNK // LANES, body, 0)

    issue_idx(0, 0, isem0)
    issue_idx(1, 1, isem1)
    issue_idx(2, 2, isem2)

    @pl.when(s < last)
    def _():
        pltpu.sync_copy(zeros_hbm.at[pl.ds(0, ROWS_A)],
                        acc_sh.at[pl.ds(row0, ROWS_A)])

    @pl.when(s == last)
    def _():
        pltpu.sync_copy(zeros_hbm, acc_sh.at[pl.ds(last * ROWS_A, ROWS_LAST)])

    wait_idx(isem0)
    pltpu.async_copy(h0_hbm.at[src_ring.at[pl.ds(0, CHUNK)]], rows.at[0], gsem0)

    plsc.subcore_barrier()

    def body(t, carry):
        i0 = t * 4
        for k in range(4):
            i = i0 + k
            p = k % 2

            wait_gather(gsems[p])

            if k == 0:
                @pl.when(i >= 1)
                def _():
                    wait_scatter(ssems[1 - p])
            else:
                wait_scatter(ssems[1 - p])

            if k == 3:
                @pl.when(i + 1 < NPT)
                def _(k=k, p=p):
                    wait_idx(isems[(k + 1) % 4])
                    pltpu.async_copy(
                        h0_hbm.at[src_ring.at[pl.ds(((k + 1) % 4) * CHUNK, CHUNK)]],
                        rows.at[1 - p], gsems[1 - p])
            else:
                wait_idx(isems[(k + 1) % 4])
                pltpu.async_copy(
                    h0_hbm.at[src_ring.at[pl.ds(((k + 1) % 4) * CHUNK, CHUNK)]],
                    rows.at[1 - p], gsems[1 - p])

            scale_rows(p, k)

            pltpu.async_copy(rows.at[p], acc_sh.at[dst_ring.at[pl.ds(k * CHUNK, CHUNK)]],
                             ssems[p], add=True)

            @pl.when(i + 3 < NPT)
            def _(k=k, i=i):
                issue_idx(i + 3, (k + 3) % 4, isems[(k + 3) % 4])

        return carry

    lax.fori_loop(0, NPT // 4, body, 0)

    wait_scatter(ssem1)
    plsc.subcore_barrier()

    @pl.when(s < last)
    def _():
        pltpu.sync_copy(acc_sh.at[pl.ds(row0, ROWS_A)],
                        out_hbm.at[c, pl.ds(row0, ROWS_A)])

    @pl.when(s == last)
    def _():
        pltpu.sync_copy(acc_sh.at[pl.ds(last * ROWS_A, ROWS_LAST)],
                        out_hbm.at[c, pl.ds(last * ROWS_A, ROWS_LAST)])


def _spmm_both(h0, dst_all, src_all, val_all, zeros):
    mesh = plsc.VectorSubcoreMesh(core_axis_name="c", subcore_axis_name="s")
    return pl.kernel(
        _spmm_body,
        out_type=jax.ShapeDtypeStruct((NUM_CORES, N_NODES, DIM), jnp.float32),
        mesh=mesh,
        scratch_types=[
            pltpu.VMEM((4 * CHUNK,), jnp.int32),
            pltpu.VMEM((4 * CHUNK,), jnp.int32),
            pltpu.VMEM((4 * CHUNK,), jnp.float32),
            pltpu.VMEM((2, CHUNK, DIM), jnp.float32),
            pltpu.VMEM_SHARED((N_NODES, DIM), jnp.float32),
            pltpu.SemaphoreType.DMA,
            pltpu.SemaphoreType.DMA,
            pltpu.SemaphoreType.DMA,
            pltpu.SemaphoreType.DMA,
            pltpu.SemaphoreType.DMA,
            pltpu.SemaphoreType.DMA,
            pltpu.SemaphoreType.DMA,
            pltpu.SemaphoreType.DMA,
        ],
    )(h0, dst_all, src_all, val_all, zeros)


def _pad_edges(a):
    return jnp.concatenate([a, jnp.zeros((E_PAD - N_EDGES,), a.dtype)])


def kernel(x, adj1_indices, adj1_values, adj2_indices, adj2_values, W1):
    h0 = _h0_matmul(x, W1)
    i1 = adj1_indices.astype(jnp.int32)
    i2 = adj2_indices.astype(jnp.int32)
    dst_all = jnp.concatenate([_pad_edges(i1[0]), _pad_edges(i2[0])])
    src_all = jnp.concatenate([_pad_edges(i1[1]), _pad_edges(i2[1])])
    val_all = jnp.concatenate([_pad_edges(adj1_values), _pad_edges(adj2_values)])
    zeros = jnp.zeros((ROWS_LAST, DIM), jnp.float32)
    hops = _spmm_both(h0, dst_all, src_all, val_all, zeros)
    return jnp.concatenate([h0, hops[0], hops[1]], axis=1)
